# Initial kernel scaffold; baseline (speedup 1.0000x reference)
import jax
import jax.numpy as jnp
from jax.experimental import pallas as pl


def kernel(x, edge_index, W1, b1, W2, b2):
    raise NotImplementedError("write your pallas kernel here")


# speedup vs baseline: 1.2249x; 1.2249x over previous
import functools

import jax
import jax.numpy as jnp
from jax import lax
from jax.experimental import pallas as pl
from jax.experimental.pallas import tpu as pltpu
from jax.experimental.pallas import tpu_sc as plsc

N_NODES_C = 10000
N_EDGES_C = 160000
H_C = 256

_NC = 2
_NS = 16
_NW = _NC * _NS
_LANES = 16

_CHUNK = 128
_EDGES_PAD = 163840
_CHUNKS_PER_W = _EDGES_PAD // _NW // _CHUNK


def _tables_body(x_ref, wa_ref, wb_ref, b1_ref, a_ref, b_ref):
    xb = x_ref[...]
    a_ref[...] = (
        jnp.dot(xb, wa_ref[...], preferred_element_type=jnp.float32) + b1_ref[...]
    )
    b_ref[...] = jnp.dot(xb, wb_ref[...], preferred_element_type=jnp.float32)


def _make_tables(x, wa, wb, b1r):
    n, h = x.shape
    blk = 1000
    grid = n // blk
    return pl.pallas_call(
        _tables_body,
        grid=(grid,),
        in_specs=[
            pl.BlockSpec((blk, h), lambda i: (i, 0)),
            pl.BlockSpec((h, h), lambda i: (0, 0)),
            pl.BlockSpec((h, h), lambda i: (0, 0)),
            pl.BlockSpec((1, h), lambda i: (0, 0)),
        ],
        out_specs=[
            pl.BlockSpec((blk, h), lambda i: (i, 0)),
            pl.BlockSpec((blk, h), lambda i: (i, 0)),
        ],
        out_shape=[
            jax.ShapeDtypeStruct((n, h), jnp.float32),
            jax.ShapeDtypeStruct((n, h), jnp.float32),
        ],
    )(x, wa, wb, b1r)


def _sc_gather_add_body(
    a_hbm, b_hbm, row_hbm, col_hbm, out_hbm, ridx, cidx, gsum, bufb, sem_a, sem_b
):
    wid = lax.axis_index("s") * _NC + lax.axis_index("c")
    base = wid * (_CHUNKS_PER_W * _CHUNK)

    def chunk_body(c, carry):
        off = base + c * _CHUNK
        pltpu.sync_copy(row_hbm.at[pl.ds(off, _CHUNK)], ridx)
        pltpu.sync_copy(col_hbm.at[pl.ds(off, _CHUNK)], cidx)
        cp_a = pltpu.async_copy(a_hbm.at[ridx], gsum, sem_a)
        cp_b = pltpu.async_copy(b_hbm.at[cidx], bufb, sem_b)
        cp_a.wait()
        cp_b.wait()

        def row_body(i, carry2):
            for j in range(H_C // _LANES):
                sl = pl.ds(j * _LANES, _LANES)
                plsc.addupdate(gsum.at[i, sl], bufb[i, sl])
            return carry2

        lax.fori_loop(0, _CHUNK, row_body, 0, unroll=False)
        pltpu.sync_copy(gsum, out_hbm.at[pl.ds(off, _CHUNK)])
        return carry

    lax.fori_loop(0, _CHUNKS_PER_W, chunk_body, 0, unroll=False)


def _make_gather_add(a, b, row_pad, col_pad):
    h = a.shape[1]
    mesh = plsc.VectorSubcoreMesh(
        core_axis_name="c", subcore_axis_name="s", num_cores=_NC, num_subcores=_NS
    )
    return pl.kernel(
        _sc_gather_add_body,
        out_type=jax.ShapeDtypeStruct((_EDGES_PAD, h), jnp.float32),
        mesh=mesh,
        scratch_types=[
            pltpu.VMEM((_CHUNK,), jnp.int32),
            pltpu.VMEM((_CHUNK,), jnp.int32),
            pltpu.VMEM((_CHUNK, h), jnp.float32),
            pltpu.VMEM((_CHUNK, h), jnp.float32),
            pltpu.SemaphoreType.DMA,
            pltpu.SemaphoreType.DMA,
        ],
    )(a, b, row_pad, col_pad)


def _mlp_body(g_ref, w2t_ref, b2_ref, o_ref):
    h = jnp.maximum(g_ref[...], 0.0)
    o_ref[...] = (
        jnp.dot(h, w2t_ref[...], preferred_element_type=jnp.float32) + b2_ref[...]
    )


def _make_mlp(g_pad, w2t, b2r, n_edges):
    h = w2t.shape[0]
    blk = 640
    grid = n_edges // blk
    return pl.pallas_call(
        _mlp_body,
        grid=(grid,),
        in_specs=[
            pl.BlockSpec((blk, h), lambda i: (i, 0)),
            pl.BlockSpec((h, h), lambda i: (0, 0)),
            pl.BlockSpec((1, h), lambda i: (0, 0)),
        ],
        out_specs=pl.BlockSpec((blk, h), lambda i: (i, 0)),
        out_shape=jax.ShapeDtypeStruct((n_edges, h), jnp.float32),
    )(g_pad, w2t, b2r)


def kernel(x, edge_index, W1, b1, W2, b2):
    n, h = x.shape
    e = edge_index.shape[1]

    row = edge_index[0].astype(jnp.int32)
    col = edge_index[1].astype(jnp.int32)
    pad = _EDGES_PAD - e
    row_pad = jnp.pad(row, (0, pad))
    col_pad = jnp.pad(col, (0, pad))

    w1t = W1.T
    wa = w1t[:h]
    wb = w1t[h:]
    w2t = W2.T
    b1r = b1.reshape(1, h)
    b2r = b2.reshape(1, h)

    a, b = _make_tables(x, wa, wb, b1r)
    g_pad = _make_gather_add(a, b, row_pad, col_pad)
    out = _make_mlp(g_pad, w2t, b2r, e)
    return out
